# baseline (speedup 1.0000x reference)
import functools

import jax
import jax.numpy as jnp
from jax import lax
from jax.experimental import pallas as pl
from jax.experimental.pallas import tpu as pltpu
from jax.experimental.pallas import tpu_sc as plsc

_NC = 2
_NS = 16
_NW = _NC * _NS
_LANE = 128
_SUB = 8


def _emb_grid(batch, seq, d_model):
    n_bg = batch // _LANE
    n_dg = d_model // _SUB
    mesh = plsc.VectorSubcoreMesh(core_axis_name="c", subcore_axis_name="s")

    @functools.partial(
        pl.kernel,
        mesh=mesh,
        out_type=jax.ShapeDtypeStruct((seq, n_dg, n_bg, _SUB, _LANE), jnp.float32),
        scratch_types=[
            pltpu.VMEM((seq, _LANE), jnp.int32),
            pltpu.VMEM((2, _LANE, d_model), jnp.float32),
            pltpu.VMEM((2, n_dg, _SUB, _LANE), jnp.float32),
            pltpu.SemaphoreType.DMA,
            pltpu.SemaphoreType.DMA,
            pltpu.SemaphoreType.DMA,
            pltpu.SemaphoreType.DMA,
        ],
        compiler_params=pltpu.CompilerParams(
            use_tc_tiling_on_sc=False, needs_layout_passes=False
        ),
    )
    def emb(idx_hbm, table_hbm, out_hbm, idx_v, rows_v, tiles_v,
            g0, g1, s0, s1):
        wid = lax.axis_index("s") * _NC + lax.axis_index("c")
        gsem = (g0, g1)
        ssem = (s0, s1)

        pltpu.sync_copy(idx_hbm.at[wid], idx_v)

        def fire_gather(s, slot):
            pltpu.async_copy(
                table_hbm.at[idx_v.at[s]], rows_v.at[slot], gsem[slot]
            )

        def wait_gather(slot):
            pltpu.make_async_copy(
                table_hbm.at[pl.ds(0, _LANE)], rows_v.at[slot], gsem[slot]
            ).wait()

        def transpose(slot):
            @plsc.parallel_loop(0, _LANE // 16, unroll=2)
            def _(j):
                rowi = lax.iota(jnp.int32, 16) + j * 16
                for dg in range(n_dg):
                    for sub in range(_SUB):
                        colv = jnp.full((16,), dg * _SUB + sub, jnp.int32)
                        v = plsc.load_gather(rows_v.at[slot], [rowi, colv])
                        tiles_v[slot, dg, sub, pl.ds(j * 16, 16)] = v

        def fire_store(s, slot):
            pltpu.async_copy(
                tiles_v.at[slot], out_hbm.at[s, :, wid], ssem[slot]
            )

        def wait_store(slot):
            pltpu.make_async_copy(
                tiles_v.at[slot], out_hbm.at[0, :, wid], ssem[slot]
            ).wait()

        fire_gather(0, 0)
        fire_gather(1, 1)
        for k in range(2):
            wait_gather(k)
            transpose(k)
            fire_gather(k + 2, k)
            fire_store(k, k)

        def superstep(t, carry):
            for b in range(2):
                k = 2 * t + b
                wait_store(b)
                wait_gather(b)
                transpose(b)
                @pl.when(k + 2 < seq)
                def _():
                    fire_gather(k + 2, b)
                fire_store(k, b)
            return carry

        lax.fori_loop(1, seq // 2, superstep, 0)
        wait_store(0)
        wait_store(1)

    return emb


def kernel(tokens, token_emb):
    b, s = tokens.shape
    v, d = token_emb.shape
    idx3 = jnp.transpose(
        tokens.reshape(_NW, _LANE, s), (0, 2, 1)
    ).astype(jnp.int32)
    out5 = _emb_grid(b, s, d)(idx3, token_emb)
    return jnp.transpose(out5, (2, 4, 0, 1, 3)).reshape(b, s, d)


# speedup vs baseline: 1.1014x; 1.1014x over previous
import functools

import jax
import jax.numpy as jnp
from jax import lax
from jax.experimental import pallas as pl
from jax.experimental.pallas import tpu as pltpu
from jax.experimental.pallas import tpu_sc as plsc

_NC = 2
_NS = 16
_NW = _NC * _NS
_LANE = 128
_SUB = 8


def _emb_grid(batch, seq, d_model):
    n_bg = batch // _LANE
    n_dg = d_model // _SUB
    mesh = plsc.VectorSubcoreMesh(core_axis_name="c", subcore_axis_name="s")

    @functools.partial(
        pl.kernel,
        mesh=mesh,
        out_type=jax.ShapeDtypeStruct((seq, n_dg, n_bg, _SUB, _LANE), jnp.float32),
        scratch_types=[
            pltpu.VMEM((seq, _LANE), jnp.int32),
            pltpu.VMEM((2, _LANE, d_model), jnp.float32),
            pltpu.VMEM((2, n_dg, _SUB, _LANE), jnp.float32),
            pltpu.SemaphoreType.DMA,
            pltpu.SemaphoreType.DMA,
            pltpu.SemaphoreType.DMA,
            pltpu.SemaphoreType.DMA,
        ],
        compiler_params=pltpu.CompilerParams(
            use_tc_tiling_on_sc=False, needs_layout_passes=False
        ),
    )
    def emb(idx_hbm, table_hbm, out_hbm, idx_v, rows_v, tiles_v,
            g0, g1, s0, s1):
        wid = lax.axis_index("s") * _NC + lax.axis_index("c")
        gsem = (g0, g1)
        ssem = (s0, s1)

        pltpu.sync_copy(idx_hbm.at[wid], idx_v)

        def fire_gather(s, slot):
            pltpu.async_copy(
                table_hbm.at[idx_v.at[s]], rows_v.at[slot], gsem[slot]
            )

        def wait_gather(slot):
            pltpu.make_async_copy(
                table_hbm.at[pl.ds(0, _LANE)], rows_v.at[slot], gsem[slot]
            ).wait()

        def transpose(slot):
            @plsc.parallel_loop(0, (_LANE // 16) * d_model, unroll=16)
            def _(i):
                j = lax.shift_right_logical(i, 6)
                c = lax.bitwise_and(i, d_model - 1)
                dg = lax.shift_right_logical(c, 3)
                sub = lax.bitwise_and(c, _SUB - 1)
                rowi = lax.iota(jnp.int32, 16) + j * 16
                colv = jnp.zeros((16,), jnp.int32) + c
                v = plsc.load_gather(rows_v.at[slot], [rowi, colv])
                tiles_v[slot, dg, sub, pl.ds(j * 16, 16)] = v

        def fire_store(s, slot):
            pltpu.async_copy(
                tiles_v.at[slot], out_hbm.at[s, :, wid], ssem[slot]
            )

        def wait_store(slot):
            pltpu.make_async_copy(
                tiles_v.at[slot], out_hbm.at[0, :, wid], ssem[slot]
            ).wait()

        fire_gather(0, 0)
        fire_gather(1, 1)
        for k in range(2):
            wait_gather(k)
            transpose(k)
            fire_gather(k + 2, k)
            fire_store(k, k)

        def superstep(t, carry):
            for b in range(2):
                k = 2 * t + b
                wait_store(b)
                wait_gather(b)
                transpose(b)
                @pl.when(k + 2 < seq)
                def _():
                    fire_gather(k + 2, b)
                fire_store(k, b)
            return carry

        lax.fori_loop(1, seq // 2, superstep, 0)
        wait_store(0)
        wait_store(1)

    return emb


def kernel(tokens, token_emb):
    b, s = tokens.shape
    v, d = token_emb.shape
    idx3 = jnp.transpose(
        tokens.reshape(_NW, _LANE, s), (0, 2, 1)
    ).astype(jnp.int32)
    out5 = _emb_grid(b, s, d)(idx3, token_emb)
    return jnp.transpose(out5, (2, 4, 0, 1, 3)).reshape(b, s, d)


# speedup vs baseline: 1.1613x; 1.0544x over previous
import functools

import jax
import jax.numpy as jnp
from jax import lax
from jax.experimental import pallas as pl
from jax.experimental.pallas import tpu as pltpu
from jax.experimental.pallas import tpu_sc as plsc

_NC = 2
_NS = 16
_NW = _NC * _NS

_X = 4


def _emb_grid(batch, seq, d_model):
    seq_per_w = batch // _NW
    n_steps = seq_per_w // _X
    n_super = n_steps // 2
    mesh = plsc.VectorSubcoreMesh(core_axis_name="c", subcore_axis_name="s")

    @functools.partial(
        pl.kernel,
        mesh=mesh,
        out_type=jax.ShapeDtypeStruct((batch, seq, d_model), jnp.float32),
        scratch_types=[
            pltpu.VMEM((seq_per_w, seq), jnp.int32),
            pltpu.VMEM((2, _X, seq, d_model), jnp.float32),
            pltpu.SemaphoreType.DMA,
            pltpu.SemaphoreType.DMA,
            pltpu.SemaphoreType.DMA,
            pltpu.SemaphoreType.DMA,
        ],
        compiler_params=pltpu.CompilerParams(use_tc_tiling_on_sc=False),
    )
    def emb(idx_hbm, table_hbm, out_hbm, idx_v, rows_v, g0, g1, s0, s1):
        wid = lax.axis_index("s") * _NC + lax.axis_index("c")
        base_seq = wid * seq_per_w
        gsem = (g0, g1)
        ssem = (s0, s1)

        pltpu.sync_copy(idx_hbm.at[pl.ds(base_seq, seq_per_w)], idx_v)

        def fire_gathers(step, slot):
            for j in range(_X):
                pltpu.async_copy(
                    table_hbm.at[idx_v.at[step * _X + j]],
                    rows_v.at[slot, j],
                    gsem[slot],
                )

        def wait_gathers(slot):
            pltpu.make_async_copy(
                out_hbm.at[pl.ds(0, _X)], rows_v.at[slot], gsem[slot]
            ).wait()

        def fire_store(step, slot):
            pltpu.async_copy(
                rows_v.at[slot],
                out_hbm.at[pl.ds(base_seq + step * _X, _X)],
                ssem[slot],
            )

        def wait_store(slot):
            pltpu.make_async_copy(
                rows_v.at[slot], out_hbm.at[pl.ds(0, _X)], ssem[slot]
            ).wait()

        fire_gathers(0, 0)
        fire_gathers(1, 1)
        wait_gathers(0)
        fire_store(0, 0)

        def superstep(t, carry):
            for b in range(2):
                k = 2 * t + b
                wait_store(b)
                fire_gathers(k, b)
                wait_gathers(1 - b)
                fire_store(k - 1, 1 - b)
            return carry

        lax.fori_loop(1, n_super, superstep, 0)

        wait_gathers(1)
        fire_store(n_steps - 1, 1)
        wait_store(0)
        wait_store(1)

    return emb


def kernel(tokens, token_emb):
    b, s = tokens.shape
    v, d = token_emb.shape
    return _emb_grid(b, s, d)(tokens.astype(jnp.int32), token_emb)
